# baseline (speedup 1.0000x reference)
import jax
import jax.numpy as jnp
from jax import lax
from jax.experimental import pallas as pl
from jax.experimental.pallas import tpu as pltpu

H = 32
W = 32
D = 128
BS = 16
RPB = 2 * D * H * W // 128
HALF = RPB // 2


def _body(row_ref, col_ref, out_ref, img):
    b = pl.program_id(0)

    @pl.when(b == 0)
    def _():
        f32 = jnp.float32
        row32 = row_ref[...]
        col32 = col_ref[...]

        s_div8 = lax.broadcasted_iota(jnp.int32, (HALF, D), 0) // 8
        cidx = lax.broadcasted_iota(jnp.int32, (HALF, D), 1)
        P = (s_div8 == cidx).astype(f32)

        dn = (((1,), (1,)), ((), ()))
        tmpc = lax.dot_general(P, col32, dn, preferred_element_type=f32)
        tmpr = lax.dot_general(P, row32, dn, preferred_element_type=f32)

        s_mod8 = lax.broadcasted_iota(jnp.int32, (HALF, H), 0) % 8
        y_div4 = lax.broadcasted_iota(jnp.int32, (HALF, H), 1) // 4
        tmpr = tmpr * (s_mod8 == y_div4).astype(f32)

        xidx = lax.broadcasted_iota(jnp.int32, (H, D), 0)
        l_mod32 = lax.broadcasted_iota(jnp.int32, (H, D), 1) % 32
        LX = (xidx == l_mod32).astype(f32)
        y_mod4 = lax.broadcasted_iota(jnp.int32, (H, D), 0) % 4
        l_div32 = lax.broadcasted_iota(jnp.int32, (H, D), 1) // 32
        LY = (y_mod4 == l_div32).astype(f32)

        dn2 = (((1,), (0,)), ((), ()))
        img[pl.ds(0, HALF), :] = lax.dot_general(
            tmpc, LX, dn2, preferred_element_type=f32
        )
        img[pl.ds(HALF, HALF), :] = lax.dot_general(
            tmpr, LY, dn2, preferred_element_type=f32
        )

    out_ref[0] = img[...]


@jax.jit
def _pos_embed(row_embed, col_embed):
    out = pl.pallas_call(
        _body,
        grid=(BS,),
        in_specs=[
            pl.BlockSpec((H, D), lambda b: (0, 0)),
            pl.BlockSpec((H, D), lambda b: (0, 0)),
        ],
        out_specs=pl.BlockSpec((1, RPB, 128), lambda b: (b, 0, 0)),
        out_shape=jax.ShapeDtypeStruct((BS, RPB, 128), jnp.float32),
        scratch_shapes=[pltpu.VMEM((RPB, 128), jnp.float32)],
    )(row_embed[:H], col_embed[:H])
    return out.reshape(BS, 2 * D, H, W)


def kernel(mask, row_embed, col_embed):
    del mask
    return _pos_embed(row_embed, col_embed)


# speedup vs baseline: 7.6466x; 7.6466x over previous
import jax
import jax.numpy as jnp
from jax.experimental import pallas as pl

H = 32
W = 32
D = 128
BS = 16


def _body(row_ref, col_ref, out_ref):
    col32 = col_ref[...]
    row32 = row_ref[...]
    colB = jnp.broadcast_to(col32[None, :, :], (H, W, D))
    rowB = jnp.broadcast_to(row32[:, None, :], (H, W, D))
    out_ref[0] = jnp.concatenate([colB, rowB], axis=-1)


@jax.jit
def _pos_embed(row_embed, col_embed):
    out = pl.pallas_call(
        _body,
        grid=(BS,),
        in_specs=[
            pl.BlockSpec((H, D), lambda b: (0, 0)),
            pl.BlockSpec((H, D), lambda b: (0, 0)),
        ],
        out_specs=pl.BlockSpec((1, H, W, 2 * D), lambda b: (b, 0, 0, 0)),
        out_shape=jax.ShapeDtypeStruct((BS, H, W, 2 * D), jnp.float32),
    )(row_embed[:H], col_embed[:H])
    return jnp.transpose(out, (0, 3, 1, 2))


def kernel(mask, row_embed, col_embed):
    del mask
    return _pos_embed(row_embed, col_embed)


# speedup vs baseline: 9.8289x; 1.2854x over previous
import jax
import jax.numpy as jnp
from jax.experimental import pallas as pl

H = 32
W = 32
D = 128
BS = 16


def _body(row_ref, col_ref, out_ref):
    col32 = col_ref[...]
    row32 = row_ref[...]
    colB = jnp.broadcast_to(col32[None, :, :], (H, W, D))
    rowB = jnp.broadcast_to(row32[:, None, :], (H, W, D))
    out_ref[0] = jnp.concatenate([colB, rowB], axis=-1)


@jax.jit
def _pos_embed(row_embed, col_embed):
    out = pl.pallas_call(
        _body,
        grid=(BS,),
        in_specs=[
            pl.BlockSpec((H, D), lambda b: (0, 0)),
            pl.BlockSpec((H, D), lambda b: (0, 0)),
        ],
        out_specs=pl.BlockSpec((1, H, W, 2 * D), lambda b: (b, 0, 0, 0)),
        out_shape=jax.ShapeDtypeStruct((BS, H, W, 2 * D), jnp.float32),
    )(row_embed, col_embed)
    return jnp.transpose(out, (0, 3, 1, 2))


def kernel(mask, row_embed, col_embed):
    del mask
    return _pos_embed(row_embed, col_embed)


# speedup vs baseline: 13.9510x; 1.4194x over previous
import jax
import jax.numpy as jnp
from jax.experimental import pallas as pl

H = 32
W = 32
D = 128
BS = 16


BPB = 4


def _body(row_ref, col_ref, out_ref):
    col32 = col_ref[...]
    row32 = row_ref[...]
    colB = jnp.broadcast_to(col32[None, :, :], (H, W, D))
    rowB = jnp.broadcast_to(row32[:, None, :], (H, W, D))
    img = jnp.concatenate([colB, rowB], axis=-1)
    out_ref[...] = jnp.broadcast_to(img[None], (BPB, H, W, 2 * D))


@jax.jit
def _pos_embed(row_embed, col_embed):
    out = pl.pallas_call(
        _body,
        grid=(BS // BPB,),
        in_specs=[
            pl.BlockSpec((H, D), lambda b: (0, 0)),
            pl.BlockSpec((H, D), lambda b: (0, 0)),
        ],
        out_specs=pl.BlockSpec((BPB, H, W, 2 * D), lambda b: (b, 0, 0, 0)),
        out_shape=jax.ShapeDtypeStruct((BS, H, W, 2 * D), jnp.float32),
    )(row_embed, col_embed)
    return jnp.transpose(out, (0, 3, 1, 2))


def kernel(mask, row_embed, col_embed):
    del mask
    return _pos_embed(row_embed, col_embed)
